# baseline (speedup 1.0000x reference)
import functools

import jax
import jax.numpy as jnp
from jax.experimental import pallas as pl
from jax.experimental.pallas import tpu as pltpu

_N_WAY = 5
_K_SHOT = 5
_TEMP = 2.0
_BIG = 1 << 20
_QPB = 3


def _proto_kernel(s_ref, sn_ref, *, b, n_shots, m, c):
    s = s_ref[...]
    s = s.reshape(b, _N_WAY, n_shots, c, m).mean(axis=2)
    sn_ref[...] = s / (jnp.sqrt(jnp.sum(s * s, axis=2, keepdims=True))
                       + 1e-8)


def _mn4_kernel(lab_ref, sn_ref, q_ref, out_ref, *, b, m, q_num, total):
    i = pl.program_id(0)

    @pl.when(i == 0)
    def _init():
        out_ref[...] = jnp.zeros_like(out_ref)

    row_i = jax.lax.broadcasted_iota(jnp.int32, (m, m), 0)
    col_i = jax.lax.broadcasted_iota(jnp.int32, (m, m), 1)
    row_f = row_i.astype(jnp.float32)
    iota_nv = jax.lax.broadcasted_iota(jnp.int32, (_N_WAY, 1), 0)

    contrib = 0.0
    for bb in range(b):
      for k in range(_QPB):
        q = q_ref[bb, k]
        q_n = q / (jnp.sqrt(jnp.sum(q * q, axis=0, keepdims=True)) + 1e-8)

        s_cls = []
        rmax = []
        for n in range(_N_WAY):
            t = jax.lax.dot_general(
                sn_ref[bb, n], q_n, (((0,), (0,)), ((), ())),
                preferred_element_type=jnp.float32)
            s_cls.append(t)
            rmax.append(jnp.max(t, axis=0, keepdims=True))

        bestv = rmax[0]
        nstar = jnp.zeros((1, m), jnp.float32)
        for n in range(1, _N_WAY):
            upd = rmax[n] > bestv
            bestv = jnp.where(upd, rmax[n], bestv)
            nstar = jnp.where(upd, float(n), nstar)

        s_best = s_cls[_N_WAY - 1]
        for n in range(_N_WAY - 2, -1, -1):
            s_best = jnp.where(nstar == float(n), s_cls[n], s_best)
        astar = jnp.min(jnp.where(s_best == bestv, row_f, float(_BIG)),
                        axis=0, keepdims=True)
        qn = nstar * float(m) + astar

        qn_t = qn.T
        bestv_t = bestv.T
        same = qn_t == qn
        stronger = (bestv_t > bestv) | ((bestv_t == bestv) & (row_i < col_i))
        dom = jnp.any(same & stronger, axis=0, keepdims=True)
        mask = jnp.where((~dom) & (bestv > -1.0), _TEMP, 0.0)

        logits = jnp.concatenate(
            [jnp.sum(r * mask, axis=1, keepdims=True) for r in rmax],
            axis=0)
        pm = jnp.max(logits, axis=0, keepdims=True)
        lse = pm + jnp.log(jnp.sum(jnp.exp(logits - pm), axis=0,
                                   keepdims=True))
        lab = lab_ref[bb * q_num + i * _QPB + k]
        picked = jnp.sum(jnp.where(iota_nv == lab, logits, 0.0), axis=0,
                         keepdims=True)
        contrib = contrib + (lse - picked) * (1.0 / total)

    out_ref[...] += contrib


def kernel(support_xf, support_y, query_xf, query_y):
    b, q_num, c, h, w = query_xf.shape
    m = h * w
    n_shots = support_xf.shape[1] // _N_WAY
    q_xf = query_xf.reshape(b, q_num, c, m)
    s_xf = support_xf.reshape(b, _N_WAY * n_shots, c, m)
    labels = query_y.reshape(b * q_num)
    total = b * q_num

    sn = pl.pallas_call(
        functools.partial(_proto_kernel, b=b, n_shots=n_shots, m=m, c=c),
        out_shape=jax.ShapeDtypeStruct((b, _N_WAY, c, m), jnp.float32),
    )(s_xf)

    body = functools.partial(_mn4_kernel, b=b, m=m, q_num=q_num, total=total)
    in_specs = [
        pl.BlockSpec(memory_space=pltpu.SMEM),
        pl.BlockSpec((b, _N_WAY, c, m), lambda i: (0, 0, 0, 0)),
        pl.BlockSpec((b, _QPB, c, m), lambda i: (0, i, 0, 0)),
    ]
    loss = pl.pallas_call(
        body,
        grid=(q_num // _QPB,),
        in_specs=in_specs,
        out_specs=pl.BlockSpec((1, 1), lambda i: (0, 0)),
        out_shape=jax.ShapeDtypeStruct((1, 1), jnp.float32),
    )(labels, sn, q_xf)
    return loss[0, 0]


# speedup vs baseline: 1.0255x; 1.0255x over previous
import functools

import jax
import jax.numpy as jnp
from jax.experimental import pallas as pl
from jax.experimental.pallas import tpu as pltpu

_N_WAY = 5
_K_SHOT = 5
_TEMP = 2.0
_BIG = 1 << 20
_QPB = 3


def _mn4_kernel(lab_ref, s_ref, q_ref, out_ref, sn_ref, *,
                b, n_shots, m, c, q_num, total):
    i = pl.program_id(0)

    @pl.when(i == 0)
    def _protos():
        s = s_ref[...]
        s = s.reshape(b, _N_WAY, n_shots, c, m).mean(axis=2)
        sn_ref[...] = s / (jnp.sqrt(jnp.sum(s * s, axis=2, keepdims=True))
                           + 1e-8)
        out_ref[...] = jnp.zeros_like(out_ref)

    row_i = jax.lax.broadcasted_iota(jnp.int32, (m, m), 0)
    col_i = jax.lax.broadcasted_iota(jnp.int32, (m, m), 1)
    row_f = row_i.astype(jnp.float32)
    iota_nv = jax.lax.broadcasted_iota(jnp.int32, (_N_WAY, 1), 0)

    contrib = 0.0
    for bb in range(b):
      for k in range(_QPB):
        q = q_ref[bb, k]
        q_n = q / (jnp.sqrt(jnp.sum(q * q, axis=0, keepdims=True)) + 1e-8)

        s_cls = []
        rmax = []
        for n in range(_N_WAY):
            t = jax.lax.dot_general(
                sn_ref[bb, n], q_n, (((0,), (0,)), ((), ())),
                preferred_element_type=jnp.float32)
            s_cls.append(t)
            rmax.append(jnp.max(t, axis=0, keepdims=True))

        bestv = rmax[0]
        nstar = jnp.zeros((1, m), jnp.float32)
        for n in range(1, _N_WAY):
            upd = rmax[n] > bestv
            bestv = jnp.where(upd, rmax[n], bestv)
            nstar = jnp.where(upd, float(n), nstar)

        s_best = s_cls[_N_WAY - 1]
        for n in range(_N_WAY - 2, -1, -1):
            s_best = jnp.where(nstar == float(n), s_cls[n], s_best)
        astar = jnp.min(jnp.where(s_best == bestv, row_f, float(_BIG)),
                        axis=0, keepdims=True)
        qn = nstar * float(m) + astar

        qn_t = qn.T
        bestv_t = bestv.T
        same = qn_t == qn
        stronger = (bestv_t > bestv) | ((bestv_t == bestv) & (row_i < col_i))
        dom = jnp.any(same & stronger, axis=0, keepdims=True)
        mask = jnp.where((~dom) & (bestv > -1.0), _TEMP, 0.0)

        logits = jnp.concatenate(
            [jnp.sum(r * mask, axis=1, keepdims=True) for r in rmax],
            axis=0)
        pm = jnp.max(logits, axis=0, keepdims=True)
        lse = pm + jnp.log(jnp.sum(jnp.exp(logits - pm), axis=0,
                                   keepdims=True))
        lab = lab_ref[bb * q_num + i * _QPB + k]
        picked = jnp.sum(jnp.where(iota_nv == lab, logits, 0.0), axis=0,
                         keepdims=True)
        contrib = contrib + (lse - picked) * (1.0 / total)

    out_ref[...] += contrib


def kernel(support_xf, support_y, query_xf, query_y):
    b, q_num, c, h, w = query_xf.shape
    m = h * w
    n_shots = support_xf.shape[1] // _N_WAY
    q_xf = query_xf.reshape(b, q_num, c, m)
    s_xf = support_xf.reshape(b, _N_WAY * n_shots, c, m)
    labels = query_y.reshape(b * q_num)
    total = b * q_num

    body = functools.partial(_mn4_kernel, b=b, n_shots=n_shots, m=m, c=c,
                             q_num=q_num, total=total)
    in_specs = [
        pl.BlockSpec(memory_space=pltpu.SMEM),
        pl.BlockSpec((b, _N_WAY * n_shots, c, m),
                     lambda i: (0, 0, 0, 0)),
        pl.BlockSpec((b, _QPB, c, m), lambda i: (0, i, 0, 0)),
    ]
    loss = pl.pallas_call(
        body,
        grid=(q_num // _QPB,),
        in_specs=in_specs,
        out_specs=pl.BlockSpec((1, 1), lambda i: (0, 0)),
        out_shape=jax.ShapeDtypeStruct((1, 1), jnp.float32),
        scratch_shapes=[pltpu.VMEM((b, _N_WAY, c, m), jnp.float32)],
    )(labels, s_xf, q_xf)
    return loss[0, 0]
